# Initial kernel scaffold; baseline (speedup 1.0000x reference)
import functools

import jax
import jax.numpy as jnp
from jax import lax
from jax.experimental import pallas as pl
from jax.experimental.pallas import tpu as pltpu
from jax.experimental.pallas import tpu_sc as plsc

NC = 2
NS = 16
LANES = 16
NW = NC * NS
CHUNK = 128


def _zero_rows(buf, nrows, d):
    def body(i, _):
        for k in range(d // LANES):
            buf[i, pl.ds(k * LANES, LANES)] = jnp.zeros((LANES,), jnp.float32)
        return 0
    lax.fori_loop(0, nrows, body, 0)


def _make_hop(n_pad, d, n_chunks):
    mesh = plsc.VectorSubcoreMesh(core_axis_name="c", subcore_axis_name="s")
    rows_per_sub = n_pad // NS
    grp = rows_per_sub // CHUNK

    @functools.partial(
        pl.kernel, mesh=mesh,
        out_type=jax.ShapeDtypeStruct((NC, n_pad, d), jnp.float32),
        scratch_types=[
            pltpu.VMEM((n_chunks, CHUNK), jnp.int32),
            pltpu.VMEM((n_chunks, CHUNK), jnp.int32),
            pltpu.VMEM((CHUNK, d), jnp.float32),
            pltpu.VMEM_SHARED((n_pad, d), jnp.float32),
            pltpu.SemaphoreType.DMA,
        ])
    def hop(y_hbm, rowi_hbm, coli_hbm, out_hbm, idxr_v, idxc_v, rows_v,
            acc_sh, sem):
        c = lax.axis_index("c")
        s = lax.axis_index("s")
        base = (c * NS + s) * n_chunks
        pltpu.sync_copy(rowi_hbm.at[pl.ds(base, n_chunks)], idxr_v)
        pltpu.sync_copy(coli_hbm.at[pl.ds(base, n_chunks)], idxc_v)
        _zero_rows(rows_v, CHUNK, d)
        for t in range(grp):
            pltpu.sync_copy(
                rows_v, acc_sh.at[pl.ds(s * rows_per_sub + t * CHUNK, CHUNK)])
        plsc.subcore_barrier()

        def body(j, _):
            pltpu.async_copy(y_hbm.at[idxr_v.at[j]], rows_v, sem).wait()
            pltpu.sync_copy(rows_v, acc_sh.at[idxc_v.at[j]], add=True)
            return 0
        lax.fori_loop(0, n_chunks, body, 0)
        plsc.subcore_barrier()
        for t in range(grp):
            off = s * rows_per_sub + t * CHUNK
            pltpu.sync_copy(acc_sh.at[pl.ds(off, CHUNK)],
                            out_hbm.at[c, pl.ds(off, CHUNK)])

    return hop


def _make_deg(n_pad, n_chunks):
    mesh = plsc.VectorSubcoreMesh(core_axis_name="c", subcore_axis_name="s")
    rows_per_sub = n_pad // NS
    grp = rows_per_sub // CHUNK

    @functools.partial(
        pl.kernel, mesh=mesh,
        out_type=jax.ShapeDtypeStruct((NC, n_pad, LANES), jnp.float32),
        scratch_types=[
            pltpu.VMEM((n_chunks, CHUNK), jnp.int32),
            pltpu.VMEM((CHUNK, LANES), jnp.float32),
            pltpu.VMEM_SHARED((n_pad, LANES), jnp.float32),
        ])
    def deg(coli_hbm, out_hbm, idxc_v, ones_v, acc_sh):
        c = lax.axis_index("c")
        s = lax.axis_index("s")
        base = (c * NS + s) * n_chunks
        pltpu.sync_copy(coli_hbm.at[pl.ds(base, n_chunks)], idxc_v)
        _zero_rows(ones_v, CHUNK, LANES)
        for t in range(grp):
            pltpu.sync_copy(
                ones_v, acc_sh.at[pl.ds(s * rows_per_sub + t * CHUNK, CHUNK)])
        def fill(i, _):
            ones_v[i, pl.ds(0, LANES)] = jnp.ones((LANES,), jnp.float32)
            return 0
        lax.fori_loop(0, CHUNK, fill, 0)
        plsc.subcore_barrier()

        def body(j, _):
            pltpu.sync_copy(ones_v, acc_sh.at[idxc_v.at[j]], add=True)
            return 0
        lax.fori_loop(0, n_chunks, body, 0)
        plsc.subcore_barrier()
        for t in range(grp):
            off = s * rows_per_sub + t * CHUNK
            pltpu.sync_copy(acc_sh.at[pl.ds(off, CHUNK)],
                            out_hbm.at[c, pl.ds(off, CHUNK)])

    return deg


_BLK = 1024


def _prep_body(degp_ref, x_ref, deg_ref, y0_ref):
    dsum = degp_ref[0] + degp_ref[1]
    deg_ref[...] = dsum
    d1 = dsum[:, 0:1]
    dinv = jnp.where(d1 > 0, lax.rsqrt(d1), 0.0)
    y0_ref[...] = x_ref[...] * dinv


def _combine_body(p_ref, deg_ref, y_ref):
    d1 = deg_ref[...][:, 0:1]
    z = p_ref[0] + p_ref[1]
    y_ref[...] = jnp.where(d1 > 0, z / d1, 0.0)


def _mm1_body(x_ref, pa_ref, pb_ref, deg_ref, w_ref, b_ref, h_ref, yh_ref):
    d1 = deg_ref[...][:, 0:1]
    dinv = jnp.where(d1 > 0, lax.rsqrt(d1), 0.0)
    x1 = (pa_ref[0] + pa_ref[1]) * dinv
    x2 = (pb_ref[0] + pb_ref[1]) * dinv
    dd = x_ref.shape[1]
    acc = jnp.dot(x_ref[...], w_ref[0:dd], preferred_element_type=jnp.float32)
    acc += jnp.dot(x1, w_ref[dd:2 * dd], preferred_element_type=jnp.float32)
    acc += jnp.dot(x2, w_ref[2 * dd:3 * dd], preferred_element_type=jnp.float32)
    h = jnp.maximum(acc + b_ref[...], 0.0)
    h_ref[...] = h
    yh_ref[...] = h * dinv


def _mm2_body(h_ref, qa_ref, qb_ref, deg_ref, w_ref, b_ref, out_ref):
    d1 = deg_ref[...][:, 0:1]
    dinv = jnp.where(d1 > 0, lax.rsqrt(d1), 0.0)
    x1 = (qa_ref[0] + qa_ref[1]) * dinv
    x2 = (qb_ref[0] + qb_ref[1]) * dinv
    hh = h_ref.shape[1]
    acc = jnp.dot(h_ref[...], w_ref[0:hh], preferred_element_type=jnp.float32)
    acc += jnp.dot(x1, w_ref[hh:2 * hh], preferred_element_type=jnp.float32)
    acc += jnp.dot(x2, w_ref[2 * hh:3 * hh], preferred_element_type=jnp.float32)
    out_ref[...] = acc + b_ref[...]


def _row_spec(d):
    return pl.BlockSpec((_BLK, d), lambda i: (i, 0))


def _pair_spec(d):
    return pl.BlockSpec((NC, _BLK, d), lambda i: (0, i, 0))


def _full_spec(shape):
    return pl.BlockSpec(shape, lambda i: tuple(0 for _ in shape))


def kernel(x, edge_index, W1, b1, W2, b2):
    n, dd = x.shape
    hdim = W1.shape[1]
    e = edge_index.shape[1]

    n_pad = -(-n // (NS * CHUNK)) * (NS * CHUNK)
    e_pad = -(-e // (NW * CHUNK)) * (NW * CHUNK)
    n_chunks = e_pad // (NW * CHUNK)
    grid = n_pad // _BLK

    row = jnp.pad(edge_index[0], (0, e_pad - e))
    col = jnp.pad(edge_index[1], (0, e_pad - e),
                  constant_values=n)
    rowi = row.reshape(NW * n_chunks, CHUNK)
    coli = col.reshape(NW * n_chunks, CHUNK)
    x_pad = jnp.pad(x, ((0, n_pad - n), (0, 0)))

    hop_d = _make_hop(n_pad, dd, n_chunks)
    hop_h = _make_hop(n_pad, hdim, n_chunks)
    degk = _make_deg(n_pad, n_chunks)

    degp = degk(coli)

    deg, y0 = pl.pallas_call(
        _prep_body,
        grid=(grid,),
        in_specs=[_pair_spec(LANES), _row_spec(dd)],
        out_specs=[_row_spec(LANES), _row_spec(dd)],
        out_shape=[jax.ShapeDtypeStruct((n_pad, LANES), jnp.float32),
                   jax.ShapeDtypeStruct((n_pad, dd), jnp.float32)],
    )(degp, x_pad)

    def combine(p, d):
        return pl.pallas_call(
            _combine_body,
            grid=(grid,),
            in_specs=[_pair_spec(d), _row_spec(LANES)],
            out_specs=_row_spec(d),
            out_shape=jax.ShapeDtypeStruct((n_pad, d), jnp.float32),
        )(p, deg)

    P1 = hop_d(y0, rowi, coli)
    y1 = combine(P1, dd)
    P2 = hop_d(y1, rowi, coli)

    h, yh = pl.pallas_call(
        _mm1_body,
        grid=(grid,),
        in_specs=[_row_spec(dd), _pair_spec(dd), _pair_spec(dd),
                  _row_spec(LANES), _full_spec(W1.shape), _full_spec((1, hdim))],
        out_specs=[_row_spec(hdim), _row_spec(hdim)],
        out_shape=[jax.ShapeDtypeStruct((n_pad, hdim), jnp.float32),
                   jax.ShapeDtypeStruct((n_pad, hdim), jnp.float32)],
    )(x_pad, P1, P2, deg, W1, b1.reshape(1, hdim))

    Q1 = hop_h(yh, rowi, coli)
    y1h = combine(Q1, hdim)
    Q2 = hop_h(y1h, rowi, coli)

    out = pl.pallas_call(
        _mm2_body,
        grid=(grid,),
        in_specs=[_row_spec(hdim), _pair_spec(hdim), _pair_spec(hdim),
                  _row_spec(LANES), _full_spec(W2.shape), _full_spec((1, dd))],
        out_specs=_row_spec(dd),
        out_shape=jax.ShapeDtypeStruct((n_pad, dd), jnp.float32),
    )(h, Q1, Q2, deg, W2, b2.reshape(1, dd))

    return out[:n]


# speedup vs baseline: 10.0260x; 10.0260x over previous
import functools

import jax
import jax.numpy as jnp
from jax import lax
from jax.experimental import pallas as pl
from jax.experimental.pallas import tpu as pltpu
from jax.experimental.pallas import tpu_sc as plsc

NC = 2
NS = 16
LANES = 16
NW = NC * NS
CHUNK = 128


def _zero_rows(buf, nrows, d):
    def body(i, _):
        for k in range(d // LANES):
            buf[i, pl.ds(k * LANES, LANES)] = jnp.zeros((LANES,), jnp.float32)
        return 0
    lax.fori_loop(0, nrows, body, 0)


def _make_hop(n_pad, d, n_chunks):
    mesh = plsc.VectorSubcoreMesh(core_axis_name="c", subcore_axis_name="s")
    rows_per_sub = n_pad // NS
    grp = rows_per_sub // CHUNK

    @functools.partial(
        pl.kernel, mesh=mesh,
        out_type=jax.ShapeDtypeStruct((NC, n_pad, d), jnp.float32),
        compiler_params=pltpu.CompilerParams(use_tc_tiling_on_sc=False),
        scratch_types=[
            pltpu.VMEM((n_chunks, CHUNK), jnp.int32),
            pltpu.VMEM((n_chunks, CHUNK), jnp.int32),
            pltpu.VMEM((CHUNK, d), jnp.float32),
            pltpu.VMEM_SHARED((n_pad, d), jnp.float32),
            pltpu.SemaphoreType.DMA,
        ])
    def hop(y_hbm, rowi_hbm, coli_hbm, out_hbm, idxr_v, idxc_v, rows_v,
            acc_sh, sem):
        c = lax.axis_index("c")
        s = lax.axis_index("s")
        wid = c * NS + s
        pltpu.sync_copy(rowi_hbm.at[wid], idxr_v)
        pltpu.sync_copy(coli_hbm.at[wid], idxc_v)
        _zero_rows(rows_v, CHUNK, d)
        for t in range(grp):
            pltpu.sync_copy(
                rows_v, acc_sh.at[pl.ds(s * rows_per_sub + t * CHUNK, CHUNK)])
        plsc.subcore_barrier()

        def body(j, _):
            pltpu.async_copy(y_hbm.at[idxr_v.at[j]], rows_v, sem).wait()
            pltpu.sync_copy(rows_v, acc_sh.at[idxc_v.at[j]], add=True)
            return 0
        lax.fori_loop(0, n_chunks, body, 0)
        plsc.subcore_barrier()
        for t in range(grp):
            off = s * rows_per_sub + t * CHUNK
            pltpu.sync_copy(acc_sh.at[pl.ds(off, CHUNK)],
                            out_hbm.at[c, pl.ds(off, CHUNK)])

    return hop


def _make_deg(n_pad, n_chunks):
    mesh = plsc.VectorSubcoreMesh(core_axis_name="c", subcore_axis_name="s")
    rows_per_sub = n_pad // NS
    grp = rows_per_sub // CHUNK

    @functools.partial(
        pl.kernel, mesh=mesh,
        out_type=jax.ShapeDtypeStruct((NC, n_pad, LANES), jnp.float32),
        compiler_params=pltpu.CompilerParams(use_tc_tiling_on_sc=False),
        scratch_types=[
            pltpu.VMEM((n_chunks, CHUNK), jnp.int32),
            pltpu.VMEM((CHUNK, LANES), jnp.float32),
            pltpu.VMEM_SHARED((n_pad, LANES), jnp.float32),
        ])
    def deg(coli_hbm, out_hbm, idxc_v, ones_v, acc_sh):
        c = lax.axis_index("c")
        s = lax.axis_index("s")
        wid = c * NS + s
        pltpu.sync_copy(coli_hbm.at[wid], idxc_v)
        _zero_rows(ones_v, CHUNK, LANES)
        for t in range(grp):
            pltpu.sync_copy(
                ones_v, acc_sh.at[pl.ds(s * rows_per_sub + t * CHUNK, CHUNK)])
        def fill(i, _):
            ones_v[i, pl.ds(0, LANES)] = jnp.ones((LANES,), jnp.float32)
            return 0
        lax.fori_loop(0, CHUNK, fill, 0)
        plsc.subcore_barrier()

        def body(j, _):
            pltpu.sync_copy(ones_v, acc_sh.at[idxc_v.at[j]], add=True)
            return 0
        lax.fori_loop(0, n_chunks, body, 0)
        plsc.subcore_barrier()
        for t in range(grp):
            off = s * rows_per_sub + t * CHUNK
            pltpu.sync_copy(acc_sh.at[pl.ds(off, CHUNK)],
                            out_hbm.at[c, pl.ds(off, CHUNK)])

    return deg


_BLK = 1024


def _prep_body(degp_ref, x_ref, deg_ref, y0_ref):
    dsum = degp_ref[0] + degp_ref[1]
    deg_ref[...] = dsum
    d1 = dsum[:, 0:1]
    dinv = jnp.where(d1 > 0, lax.rsqrt(d1), 0.0)
    y0_ref[...] = x_ref[...] * dinv


def _combine_body(p_ref, deg_ref, y_ref):
    d1 = deg_ref[...][:, 0:1]
    z = p_ref[0] + p_ref[1]
    y_ref[...] = jnp.where(d1 > 0, z / d1, 0.0)


def _mm1_body(x_ref, pa_ref, pb_ref, deg_ref, w_ref, b_ref, h_ref, yh_ref):
    d1 = deg_ref[...][:, 0:1]
    dinv = jnp.where(d1 > 0, lax.rsqrt(d1), 0.0)
    x1 = (pa_ref[0] + pa_ref[1]) * dinv
    x2 = (pb_ref[0] + pb_ref[1]) * dinv
    dd = x_ref.shape[1]
    acc = jnp.dot(x_ref[...], w_ref[0:dd], preferred_element_type=jnp.float32)
    acc += jnp.dot(x1, w_ref[dd:2 * dd], preferred_element_type=jnp.float32)
    acc += jnp.dot(x2, w_ref[2 * dd:3 * dd], preferred_element_type=jnp.float32)
    h = jnp.maximum(acc + b_ref[...], 0.0)
    h_ref[...] = h
    yh_ref[...] = h * dinv


def _mm2_body(h_ref, qa_ref, qb_ref, deg_ref, w_ref, b_ref, out_ref):
    d1 = deg_ref[...][:, 0:1]
    dinv = jnp.where(d1 > 0, lax.rsqrt(d1), 0.0)
    x1 = (qa_ref[0] + qa_ref[1]) * dinv
    x2 = (qb_ref[0] + qb_ref[1]) * dinv
    hh = h_ref.shape[1]
    acc = jnp.dot(h_ref[...], w_ref[0:hh], preferred_element_type=jnp.float32)
    acc += jnp.dot(x1, w_ref[hh:2 * hh], preferred_element_type=jnp.float32)
    acc += jnp.dot(x2, w_ref[2 * hh:3 * hh], preferred_element_type=jnp.float32)
    out_ref[...] = acc + b_ref[...]


def _row_spec(d):
    return pl.BlockSpec((_BLK, d), lambda i: (i, 0))


def _pair_spec(d):
    return pl.BlockSpec((NC, _BLK, d), lambda i: (0, i, 0))


def _full_spec(shape):
    return pl.BlockSpec(shape, lambda i: tuple(0 for _ in shape))


def kernel(x, edge_index, W1, b1, W2, b2):
    n, dd = x.shape
    hdim = W1.shape[1]
    e = edge_index.shape[1]

    n_pad = -(-n // (NS * CHUNK)) * (NS * CHUNK)
    e_pad = -(-e // (NW * CHUNK)) * (NW * CHUNK)
    n_chunks = e_pad // (NW * CHUNK)
    grid = n_pad // _BLK

    row = jnp.pad(edge_index[0], (0, e_pad - e))
    col = jnp.pad(edge_index[1], (0, e_pad - e),
                  constant_values=n)
    rowi = row.reshape(NW, n_chunks, CHUNK)
    coli = col.reshape(NW, n_chunks, CHUNK)
    x_pad = jnp.pad(x, ((0, n_pad - n), (0, 0)))

    hop_d = _make_hop(n_pad, dd, n_chunks)
    hop_h = _make_hop(n_pad, hdim, n_chunks)
    degk = _make_deg(n_pad, n_chunks)

    degp = degk(coli)

    deg, y0 = pl.pallas_call(
        _prep_body,
        grid=(grid,),
        in_specs=[_pair_spec(LANES), _row_spec(dd)],
        out_specs=[_row_spec(LANES), _row_spec(dd)],
        out_shape=[jax.ShapeDtypeStruct((n_pad, LANES), jnp.float32),
                   jax.ShapeDtypeStruct((n_pad, dd), jnp.float32)],
    )(degp, x_pad)

    def combine(p, d):
        return pl.pallas_call(
            _combine_body,
            grid=(grid,),
            in_specs=[_pair_spec(d), _row_spec(LANES)],
            out_specs=_row_spec(d),
            out_shape=jax.ShapeDtypeStruct((n_pad, d), jnp.float32),
        )(p, deg)

    P1 = hop_d(y0, rowi, coli)
    y1 = combine(P1, dd)
    P2 = hop_d(y1, rowi, coli)

    h, yh = pl.pallas_call(
        _mm1_body,
        grid=(grid,),
        in_specs=[_row_spec(dd), _pair_spec(dd), _pair_spec(dd),
                  _row_spec(LANES), _full_spec(W1.shape), _full_spec((1, hdim))],
        out_specs=[_row_spec(hdim), _row_spec(hdim)],
        out_shape=[jax.ShapeDtypeStruct((n_pad, hdim), jnp.float32),
                   jax.ShapeDtypeStruct((n_pad, hdim), jnp.float32)],
    )(x_pad, P1, P2, deg, W1, b1.reshape(1, hdim))

    Q1 = hop_h(yh, rowi, coli)
    y1h = combine(Q1, hdim)
    Q2 = hop_h(y1h, rowi, coli)

    out = pl.pallas_call(
        _mm2_body,
        grid=(grid,),
        in_specs=[_row_spec(hdim), _pair_spec(hdim), _pair_spec(hdim),
                  _row_spec(LANES), _full_spec(W2.shape), _full_spec((1, dd))],
        out_specs=_row_spec(dd),
        out_shape=jax.ShapeDtypeStruct((n_pad, dd), jnp.float32),
    )(h, Q1, Q2, deg, W2, b2.reshape(1, dd))

    return out[:n]
